# baseline (speedup 1.0000x reference)
import functools

import jax
import jax.numpy as jnp
from jax import lax
from jax.experimental import pallas as pl
from jax.experimental.pallas import tpu as pltpu
from jax.experimental.pallas import tpu_sc as plsc

B, N, L, D = 4, 8192, 64, 768
BN = 2048
NB = N // BN
VK = 32
TK = 5


def _bf16x3_nt(a, b_mat):
    a_hi = a.astype(jnp.bfloat16).astype(jnp.float32)
    a_lo = a - a_hi
    b_hi = b_mat.astype(jnp.bfloat16).astype(jnp.float32)
    b_lo = b_mat - b_hi
    dn = (((1,), (1,)), ((), ()))
    return (lax.dot_general(a_hi, b_hi, dn, preferred_element_type=jnp.float32)
            + lax.dot_general(a_hi, b_lo, dn, preferred_element_type=jnp.float32)
            + lax.dot_general(a_lo, b_hi, dn, preferred_element_type=jnp.float32))


def _fused_kernel(mask_ref, text_ref, vision_ref, gt_ref, idx_ref,
                  ntxt_ref, acc_ref, s_ref, ptxt_ref,
                  acc_all_ref, s_all_ref, ptxt_all_ref, pv_all_ref):
    b = pl.program_id(0)
    nb = pl.program_id(1)

    @pl.when(nb == 0)
    def _init():
        t = text_ref[0]
        nrm = jnp.sqrt(jnp.sum(t * t, axis=1, keepdims=True))
        ntxt_ref[...] = t / jnp.maximum(nrm, 1e-12)
        acc_ref[...] = jnp.zeros_like(acc_ref)
        s_ref[...] = jnp.zeros_like(s_ref)
        ptxt_ref[...] = jnp.zeros_like(ptxt_ref)

    v = vision_ref[0]
    n2 = jnp.sum(v * v, axis=1, keepdims=True)
    inv_vn = 1.0 / jnp.maximum(jnp.sqrt(n2), 1e-12)
    nv = v * inv_vn

    sim = _bf16x3_nt(ntxt_ref[...], nv)
    mask = mask_ref[0]
    sim = jnp.where(mask > 0.0, sim, -1.0)

    pv_all_ref[pl.ds(b * NB + nb, 1), :] = jnp.mean(sim, axis=0, keepdims=True)

    ptxt_ref[...] += jnp.sum(sim, axis=1, keepdims=True)

    p = jnp.exp(sim)
    s_ref[...] += jnp.sum(p, axis=1, keepdims=True)
    acc_ref[...] += lax.dot_general(
        p, v, (((1,), (0,)), ((), ())), preferred_element_type=jnp.float32)

    @pl.when(nb == NB - 1)
    def _stash():
        acc_all_ref[b] = acc_ref[...]
        s_all_ref[b] = s_ref[...]
        ptxt_all_ref[b] = ptxt_ref[...]

    @pl.when((b == B - 1) & (nb == NB - 1))
    def _finalize():
        l_iota = lax.broadcasted_iota(jnp.int32, (L, 1), 0).astype(jnp.float32)
        r_i = lax.broadcasted_iota(jnp.int32, (L, L), 0).astype(jnp.float32)
        c_i = lax.broadcasted_iota(jnp.int32, (L, L), 1).astype(jnp.float32)
        strict_lower = jnp.where(c_i < r_i, 1.0, 0.0)
        k_row = lax.broadcasted_iota(jnp.int32, (L, TK), 1).astype(jnp.float32)
        g_iota = (lax.broadcasted_iota(jnp.int32, (NB, BN), 0) * BN +
                  lax.broadcasted_iota(jnp.int32, (NB, BN), 1)
                  ).astype(jnp.float32)
        k_c = lax.broadcasted_iota(jnp.int32, (VK, 1), 0).astype(jnp.float32)
        k_r = lax.broadcasted_iota(jnp.int32, (1, VK), 1).astype(jnp.float32)

        for fb in range(B):
            pt = ptxt_all_ref[fb] / N
            sel_mask = jnp.zeros((L, 1), jnp.float32)
            vals = pt
            for _ in range(TK):
                mx = jnp.max(vals)
                pos = jnp.sum(jnp.where(vals == mx, l_iota, 0.0))
                sel_mask = sel_mask + jnp.where(l_iota == pos, 1.0, 0.0)
                vals = jnp.where(l_iota == pos, -jnp.inf, vals)
            rank_l = lax.dot_general(
                strict_lower, sel_mask, (((1,), (0,)), ((), ())),
                preferred_element_type=jnp.float32)
            onehot = jnp.where(
                (sel_mask > 0.0) & (rank_l == k_row), 1.0, 0.0)
            weighted = acc_all_ref[fb] / s_all_ref[fb]
            gt = lax.dot_general(
                onehot, weighted, (((0,), (0,)), ((), ())),
                preferred_element_type=jnp.float32)
            gt_ref[fb] = gt

            vvals = pv_all_ref[fb * NB:(fb + 1) * NB, :]
            lo = jnp.full((1, 1), -1.5, jnp.float32)
            hi = jnp.full((1, 1), 1.5, jnp.float32)
            for _ in range(40):
                mid = 0.5 * (lo + hi)
                cnt = jnp.sum(jnp.where(vvals >= mid, 1.0, 0.0),
                              axis=(0, 1), keepdims=True)
                ok = cnt >= VK
                lo = jnp.where(ok, mid, lo)
                hi = jnp.where(ok, hi, mid)
            sel = jnp.where(vvals >= lo, 1.0, 0.0)
            lane_pos = lax.broadcasted_iota(jnp.int32, (NB, BN), 1)
            pfx = sel
            sh = 1
            while sh < BN:
                pfx = pfx + jnp.where(lane_pos >= sh,
                                      pltpu.roll(pfx, sh, axis=1), 0.0)
                sh *= 2
            row_pos = lax.broadcasted_iota(jnp.int32, (NB, 1), 0)
            totals = pfx[:, BN - 1:BN]
            offs = jnp.zeros((NB, 1), jnp.float32)
            for k in range(1, NB):
                offs = offs + jnp.where(row_pos >= k,
                                        pltpu.roll(totals, k, axis=0), 0.0)
            rank_incl = pfx + offs
            idxcol = jnp.zeros((VK, 1), jnp.float32)
            for j in range(VK):
                cj = jnp.sum(jnp.where(rank_incl <= j, 1.0, 0.0),
                             axis=(0, 1), keepdims=True)
                idxcol = idxcol + jnp.where(k_c == j, cj, 0.0)
            idx_ref[fb] = idxcol.astype(jnp.int32) + fb * N


_SC_ROWS = B * VK
_SC_W = 16
_SC_PER_W = _SC_ROWS // _SC_W


@functools.cache
def _sc_gather_fn():
    @functools.partial(
        pl.kernel,
        mesh=plsc.VectorSubcoreMesh(core_axis_name="c", subcore_axis_name="s"),
        out_type=jax.ShapeDtypeStruct((_SC_ROWS, D), jnp.float32),
        scratch_types=[
            pltpu.VMEM((_SC_PER_W,), jnp.int32),
            pltpu.VMEM((_SC_PER_W, D), jnp.float32),
            pltpu.SemaphoreType.DMA,
        ],
    )
    def _sc_gather(table_hbm, idx_hbm, out_hbm, idx_v, rows_v, sem):
        wid = lax.axis_index("s") * 2 + lax.axis_index("c")

        @pl.when(wid < _SC_W)
        def _():
            base = wid * _SC_PER_W
            pltpu.sync_copy(idx_hbm.at[pl.ds(base, _SC_PER_W)], idx_v)
            pltpu.async_copy(table_hbm.at[idx_v], rows_v, sem).wait()
            pltpu.sync_copy(rows_v, out_hbm.at[pl.ds(base, _SC_PER_W)])

    return _sc_gather


@jax.jit
def kernel(vision_embedding, text_embedding, attention_mask):
    mask_f = attention_mask.astype(jnp.float32).reshape(B, L, 1)

    gt, idx = pl.pallas_call(
        _fused_kernel,
        grid=(B, NB),
        in_specs=[
            pl.BlockSpec((1, L, 1), lambda b, nb: (b, 0, 0)),
            pl.BlockSpec((1, L, D), lambda b, nb: (b, 0, 0)),
            pl.BlockSpec((1, BN, D), lambda b, nb: (b, nb, 0)),
        ],
        out_specs=[
            pl.BlockSpec((B, TK, D), lambda b, nb: (0, 0, 0)),
            pl.BlockSpec((B, VK, 1), lambda b, nb: (0, 0, 0)),
        ],
        out_shape=[
            jax.ShapeDtypeStruct((B, TK, D), jnp.float32),
            jax.ShapeDtypeStruct((B, VK, 1), jnp.int32),
        ],
        scratch_shapes=[
            pltpu.VMEM((L, D), jnp.float32),
            pltpu.VMEM((L, D), jnp.float32),
            pltpu.VMEM((L, 1), jnp.float32),
            pltpu.VMEM((L, 1), jnp.float32),
            pltpu.VMEM((B, L, D), jnp.float32),
            pltpu.VMEM((B, L, 1), jnp.float32),
            pltpu.VMEM((B, L, 1), jnp.float32),
            pltpu.VMEM((B * NB, BN), jnp.float32),
        ],
        compiler_params=pltpu.CompilerParams(
            dimension_semantics=("arbitrary", "arbitrary")),
    )(mask_f, text_embedding, vision_embedding)

    flat_idx = idx.reshape(B * VK)
    gv = _sc_gather_fn()(vision_embedding.reshape(B * N, D), flat_idx)

    return jnp.concatenate([gt, gv.reshape(B, VK, D)], axis=1)


# speedup vs baseline: 1.1863x; 1.1863x over previous
import functools

import jax
import jax.numpy as jnp
from jax import lax
from jax.experimental import pallas as pl
from jax.experimental.pallas import tpu as pltpu
from jax.experimental.pallas import tpu_sc as plsc

B, N, L, D = 4, 8192, 64, 768
BN = 2048
NB = N // BN
VK = 32
TK = 5


def _bf16x3_nt(a, b_mat):
    a_hi = a.astype(jnp.bfloat16).astype(jnp.float32)
    a_lo = a - a_hi
    b_hi = b_mat.astype(jnp.bfloat16).astype(jnp.float32)
    b_lo = b_mat - b_hi
    dn = (((1,), (1,)), ((), ()))
    return (lax.dot_general(a_hi, b_hi, dn, preferred_element_type=jnp.float32)
            + lax.dot_general(a_hi, b_lo, dn, preferred_element_type=jnp.float32)
            + lax.dot_general(a_lo, b_hi, dn, preferred_element_type=jnp.float32))


def _fused_kernel(mask_ref, text_ref, vision_ref, gt_ref, idx_ref,
                  ntxt_ref, acc_ref, s_ref, ptxt_ref,
                  acc_all_ref, s_all_ref, ptxt_all_ref, pv_all_ref):
    b = pl.program_id(0)
    nb = pl.program_id(1)

    @pl.when(nb == 0)
    def _init():
        t = text_ref[0]
        nrm = jnp.sqrt(jnp.sum(t * t, axis=1, keepdims=True))
        ntxt_ref[...] = t / jnp.maximum(nrm, 1e-12)
        acc_ref[...] = jnp.zeros_like(acc_ref)
        s_ref[...] = jnp.zeros_like(s_ref)
        ptxt_ref[...] = jnp.zeros_like(ptxt_ref)

    v = vision_ref[0]
    n2 = jnp.sum(v * v, axis=1, keepdims=True)
    inv_vn = 1.0 / jnp.maximum(jnp.sqrt(n2), 1e-12)
    nv = v * inv_vn

    sim = _bf16x3_nt(ntxt_ref[...], nv)
    mask = mask_ref[0]
    sim = jnp.where(mask > 0.0, sim, -1.0)

    pv_all_ref[pl.ds(b * NB + nb, 1), :] = jnp.mean(sim, axis=0, keepdims=True)

    ptxt_ref[...] += jnp.sum(sim, axis=1, keepdims=True)

    p = jnp.exp(sim)
    s_ref[...] += jnp.sum(p, axis=1, keepdims=True)
    acc_ref[...] += lax.dot_general(
        p, v, (((1,), (0,)), ((), ())), preferred_element_type=jnp.float32)

    @pl.when(nb == NB - 1)
    def _stash():
        acc_all_ref[b] = acc_ref[...]
        s_all_ref[b] = s_ref[...]
        ptxt_all_ref[b] = ptxt_ref[...]

    @pl.when((b == B - 1) & (nb == NB - 1))
    def _finalize():
        l_iota = lax.broadcasted_iota(jnp.int32, (L, 1), 0).astype(jnp.float32)
        r_i = lax.broadcasted_iota(jnp.int32, (L, L), 0).astype(jnp.float32)
        c_i = lax.broadcasted_iota(jnp.int32, (L, L), 1).astype(jnp.float32)
        strict_lower = jnp.where(c_i < r_i, 1.0, 0.0)
        k_row = lax.broadcasted_iota(jnp.int32, (L, TK), 1).astype(jnp.float32)
        g_iota = (lax.broadcasted_iota(jnp.int32, (NB, BN), 0) * BN +
                  lax.broadcasted_iota(jnp.int32, (NB, BN), 1)
                  ).astype(jnp.float32)
        k_c = lax.broadcasted_iota(jnp.int32, (VK, 1), 0).astype(jnp.float32)
        k_r = lax.broadcasted_iota(jnp.int32, (1, VK), 1).astype(jnp.float32)

        for fb in range(B):
            pt = ptxt_all_ref[fb] / N
            sel_mask = jnp.zeros((L, 1), jnp.float32)
            vals = pt
            for _ in range(TK):
                mx = jnp.max(vals)
                pos = jnp.sum(jnp.where(vals == mx, l_iota, 0.0))
                sel_mask = sel_mask + jnp.where(l_iota == pos, 1.0, 0.0)
                vals = jnp.where(l_iota == pos, -jnp.inf, vals)
            rank_l = lax.dot_general(
                strict_lower, sel_mask, (((1,), (0,)), ((), ())),
                preferred_element_type=jnp.float32)
            onehot = jnp.where(
                (sel_mask > 0.0) & (rank_l == k_row), 1.0, 0.0)
            weighted = acc_all_ref[fb] / s_all_ref[fb]
            gt = lax.dot_general(
                onehot, weighted, (((0,), (0,)), ((), ())),
                preferred_element_type=jnp.float32)
            gt_ref[fb] = gt

        pv4 = jnp.concatenate([
            jnp.concatenate(
                [pv_all_ref[fb * NB + k:fb * NB + k + 1, :] for k in range(NB)],
                axis=1)
            for fb in range(B)], axis=0)
        lo = jnp.full((B, 1), -1.5, jnp.float32)
        hi = jnp.full((B, 1), 1.5, jnp.float32)
        for _ in range(40):
            mid = 0.5 * (lo + hi)
            cnt = jnp.sum(jnp.where(pv4 >= mid, 1.0, 0.0),
                          axis=1, keepdims=True)
            ok = cnt >= VK
            lo = jnp.where(ok, mid, lo)
            hi = jnp.where(ok, hi, mid)
        sel = jnp.where(pv4 >= lo, 1.0, 0.0)
        lane_pos = lax.broadcasted_iota(jnp.int32, (B, N), 1)
        pfx = sel
        sh = 1
        while sh < N:
            pfx = pfx + jnp.where(lane_pos >= sh,
                                  pltpu.roll(pfx, sh, axis=1), 0.0)
            sh *= 2
        k_vk = lax.broadcasted_iota(jnp.int32, (1, VK), 1).astype(jnp.float32)
        idxmat = jnp.zeros((B, VK), jnp.float32)
        for j in range(VK):
            cj = jnp.sum(jnp.where(pfx <= j, 1.0, 0.0),
                         axis=1, keepdims=True)
            idxmat = idxmat + jnp.where(k_vk == j, cj, 0.0)
        brow = lax.broadcasted_iota(jnp.int32, (B, VK), 0).astype(jnp.float32)
        idx_ref[...] = (idxmat + brow * N).astype(jnp.int32).reshape(B, 1, VK)


_SC_ROWS = B * VK
_SC_W = 16
_SC_PER_W = _SC_ROWS // _SC_W


@functools.cache
def _sc_gather_fn():
    @functools.partial(
        pl.kernel,
        mesh=plsc.VectorSubcoreMesh(core_axis_name="c", subcore_axis_name="s"),
        out_type=jax.ShapeDtypeStruct((_SC_ROWS, D), jnp.float32),
        scratch_types=[
            pltpu.VMEM((_SC_PER_W,), jnp.int32),
            pltpu.VMEM((_SC_PER_W, D), jnp.float32),
            pltpu.SemaphoreType.DMA,
        ],
    )
    def _sc_gather(table_hbm, idx_hbm, out_hbm, idx_v, rows_v, sem):
        wid = lax.axis_index("s") * 2 + lax.axis_index("c")

        @pl.when(wid < _SC_W)
        def _():
            base = wid * _SC_PER_W
            pltpu.sync_copy(idx_hbm.at[pl.ds(base, _SC_PER_W)], idx_v)
            pltpu.async_copy(table_hbm.at[idx_v], rows_v, sem).wait()
            pltpu.sync_copy(rows_v, out_hbm.at[pl.ds(base, _SC_PER_W)])

    return _sc_gather


@jax.jit
def kernel(vision_embedding, text_embedding, attention_mask):
    mask_f = attention_mask.astype(jnp.float32).reshape(B, L, 1)

    gt, idx = pl.pallas_call(
        _fused_kernel,
        grid=(B, NB),
        in_specs=[
            pl.BlockSpec((1, L, 1), lambda b, nb: (b, 0, 0)),
            pl.BlockSpec((1, L, D), lambda b, nb: (b, 0, 0)),
            pl.BlockSpec((1, BN, D), lambda b, nb: (b, nb, 0)),
        ],
        out_specs=[
            pl.BlockSpec((B, TK, D), lambda b, nb: (0, 0, 0)),
            pl.BlockSpec((B, 1, VK), lambda b, nb: (0, 0, 0)),
        ],
        out_shape=[
            jax.ShapeDtypeStruct((B, TK, D), jnp.float32),
            jax.ShapeDtypeStruct((B, 1, VK), jnp.int32),
        ],
        scratch_shapes=[
            pltpu.VMEM((L, D), jnp.float32),
            pltpu.VMEM((L, D), jnp.float32),
            pltpu.VMEM((L, 1), jnp.float32),
            pltpu.VMEM((L, 1), jnp.float32),
            pltpu.VMEM((B, L, D), jnp.float32),
            pltpu.VMEM((B, L, 1), jnp.float32),
            pltpu.VMEM((B, L, 1), jnp.float32),
            pltpu.VMEM((B * NB, BN), jnp.float32),
        ],
        compiler_params=pltpu.CompilerParams(
            dimension_semantics=("arbitrary", "arbitrary")),
    )(mask_f, text_embedding, vision_embedding)

    flat_idx = idx.reshape(B * VK)
    gv = _sc_gather_fn()(vision_embedding.reshape(B * N, D), flat_idx)

    return jnp.concatenate([gt, gv.reshape(B, VK, D)], axis=1)
